# baseline (speedup 1.0000x reference)
import functools

import jax
import jax.numpy as jnp
from jax import lax
from jax.experimental import pallas as pl
from jax.experimental.pallas import tpu as pltpu
from jax.experimental.pallas import tpu_sc as plsc

_B, _T, _DIM = 2, 2048, 768
_FF = 3072
_E = 8
_TOPK = 2
_N = _B * _T
_S = _N * _TOPK
_CHUNK = _S // _E
_NW = 32
_PER_W = _S // _NW
_TOK_W = _N // _NW
_FT = 2
_FTS = _FF // _FT


def _router_body(x_ref, rw_ref, idx_ref, wts_ref, base_ref):
    x = x_ref[...]
    rw = rw_ref[...]
    logits = lax.dot_general(x, rw, (((1,), (1,)), ((), ())),
                             preferred_element_type=jnp.float32)
    col = lax.broadcasted_iota(jnp.int32, (_N, _E), 1)
    m1 = jnp.max(logits, axis=1, keepdims=True)
    i1 = jnp.min(jnp.where(logits == m1, col, _E), axis=1, keepdims=True)
    masked = jnp.where(col == i1, -jnp.inf, logits)
    m2 = jnp.max(masked, axis=1, keepdims=True)
    i2 = jnp.min(jnp.where(masked == m2, col, _E), axis=1, keepdims=True)
    e21 = jnp.exp(m2 - m1)
    denom = 1.0 + e21
    idx_ref[...] = jnp.concatenate([i1, i2], axis=1)
    wts_ref[...] = jnp.concatenate([1.0 / denom, e21 / denom], axis=1)

    m_hits = ((i1 == col).astype(jnp.float32)
              + (i2 == col).astype(jnp.float32))
    r_row = lax.broadcasted_iota(jnp.int32, (_NW, _N), 0)
    r_tok = lax.broadcasted_iota(jnp.int32, (_NW, _N), 1)
    rsel = (r_tok // _TOK_W == r_row).astype(jnp.float32)
    counts = jnp.dot(rsel, m_hits, preferred_element_type=jnp.float32)
    tri_a = lax.broadcasted_iota(jnp.int32, (_NW, _NW), 0)
    tri_b = lax.broadcasted_iota(jnp.int32, (_NW, _NW), 1)
    tril_strict = (tri_b < tri_a).astype(jnp.float32)
    prefix = jnp.dot(tril_strict, counts, preferred_element_type=jnp.float32,
                     precision=lax.Precision.HIGHEST)
    totals = jnp.sum(counts, axis=0, keepdims=True)
    u_a = lax.broadcasted_iota(jnp.int32, (_E, _E), 0)
    u_b = lax.broadcasted_iota(jnp.int32, (_E, _E), 1)
    u_strict = (u_a < u_b).astype(jnp.float32)
    offs = jnp.dot(totals, u_strict, preferred_element_type=jnp.float32,
                   precision=lax.Precision.HIGHEST)
    base = (prefix + offs).astype(jnp.int32)
    base_ref[...] = jnp.concatenate(
        [base, jnp.zeros((_NW, _E), jnp.int32)], axis=1)


def _router(x_flat, router_W):
    return pl.pallas_call(
        _router_body,
        out_shape=(
            jax.ShapeDtypeStruct((_N, _TOPK), jnp.int32),
            jax.ShapeDtypeStruct((_N, _TOPK), jnp.float32),
            jax.ShapeDtypeStruct((_NW, 16), jnp.int32),
        ),
    )(x_flat, router_W)


def _rank_body(ids_hbm, base_hbm, pos_hbm, ids_v, base_v, pos_stage):
    nc = 2
    wid = lax.axis_index("s") * nc + lax.axis_index("c")
    pltpu.sync_copy(ids_hbm.at[pl.ds(wid * _PER_W, _PER_W)], ids_v)
    pltpu.sync_copy(base_hbm.at[wid], base_v)
    cnt = [jnp.int32(0)] * _E
    for v in range(_PER_W // 16):
        iv = ids_v[pl.ds(v * 16, 16)]
        rank = jnp.zeros((16,), jnp.int32)
        for e in range(_E):
            m = iv == e
            ones = jnp.where(m, 1, 0).astype(jnp.int32)
            csum = plsc.cumsum(ones)
            rank = jnp.where(m, csum - 1 + cnt[e], rank)
            cnt[e] = cnt[e] + jnp.sum(ones)
        pos_stage[pl.ds(v * 16, 16)] = plsc.load_gather(base_v, [iv]) + rank
    pltpu.sync_copy(pos_stage, pos_hbm.at[pl.ds(wid * _PER_W, _PER_W)])


def _rank(ids_flat, base):
    mesh = plsc.VectorSubcoreMesh(core_axis_name="c", subcore_axis_name="s")
    fn = pl.kernel(
        _rank_body,
        compiler_params=pltpu.CompilerParams(needs_layout_passes=False),
        out_type=jax.ShapeDtypeStruct((_S,), jnp.int32),
        mesh=mesh,
        scratch_types=[
            pltpu.VMEM((_PER_W,), jnp.int32),
            pltpu.VMEM((16,), jnp.int32),
            pltpu.VMEM((_PER_W,), jnp.int32),
        ],
    )
    return fn(ids_flat, base)


def _gather_sorted_body(x_hbm, pos_hbm, perm_hbm,
                        pos_all, src_all, src_0, src_1, src_2, src_3,
                        rows_a, rows_b, gsem, ssem):
    nc = 2
    wid = lax.axis_index("s") * nc + lax.axis_index("c")
    src_refs = [src_0, src_1, src_2, src_3]
    pltpu.sync_copy(pos_hbm, pos_all)
    iota16 = lax.broadcasted_iota(jnp.int32, (16,), 0)
    lo = wid * _PER_W

    def scan(q, _):
        pos16 = plsc.load_gather(pos_all, [q * 16 + iota16])
        tok16 = lax.shift_right_arithmetic(q * 16 + iota16, 1)
        plsc.store_scatter(src_all, [pos16], tok16)
        return 0

    lax.fori_loop(0, _S // 16, scan, 0)
    for cc in range(4):
        for k in range(4):
            idxs = lo + cc * 64 + k * 16 + iota16
            vals = plsc.load_gather(src_all, [idxs])
            src_refs[cc][pl.ds(k * 16, 16)] = jnp.clip(vals, 0, _N - 1)
    nchunk = _PER_W // 64
    g = pltpu.async_copy(x_hbm.at[src_refs[0]], rows_a, gsem)
    for c in range(nchunk):
        buf = rows_a if c % 2 == 0 else rows_b
        g.wait()
        if c + 1 < nchunk:
            nbuf = rows_b if c % 2 == 0 else rows_a
            g = pltpu.async_copy(x_hbm.at[src_refs[c + 1]], nbuf, gsem)
        s = pltpu.async_copy(buf, perm_hbm.at[pl.ds(lo + c * 64, 64)], ssem)
        s.wait()


def _gather_sorted(x_flat, pos):
    mesh = plsc.VectorSubcoreMesh(core_axis_name="c", subcore_axis_name="s")
    fn = pl.kernel(
        _gather_sorted_body,
        compiler_params=pltpu.CompilerParams(needs_layout_passes=False),
        out_type=jax.ShapeDtypeStruct((_S, _DIM // 2), jnp.int32),
        mesh=mesh,
        scratch_types=[
            pltpu.VMEM((_S,), jnp.int32),
            pltpu.VMEM((_S,), jnp.int32),
            pltpu.VMEM((64,), jnp.int32),
            pltpu.VMEM((64,), jnp.int32),
            pltpu.VMEM((64,), jnp.int32),
            pltpu.VMEM((64,), jnp.int32),
            pltpu.VMEM((64, _DIM // 2), jnp.int32),
            pltpu.VMEM((64, _DIM // 2), jnp.int32),
            pltpu.SemaphoreType.DMA,
            pltpu.SemaphoreType.DMA,
        ],
    )
    return fn(x_flat, pos)


def _ffn_body(xc_ref, w1_ref, w2_ref, o_ref):
    f = pl.program_id(1)
    xb = xc_ref[...]
    w1b = w1_ref[0]
    h = lax.dot_general(xb, w1b, (((1,), (1,)), ((), ())),
                        preferred_element_type=jnp.float32)
    h = 0.5 * h * (1.0 + jnp.tanh(0.7978845608028654 * (h + 0.044715 * h * h * h)))
    w2b = w2_ref[0]
    contrib = lax.dot_general(h.astype(jnp.bfloat16), w2b,
                              (((1,), (1,)), ((), ())),
                              preferred_element_type=jnp.float32)

    @pl.when(f == 0)
    def _init():
        o_ref[...] = contrib

    @pl.when(f != 0)
    def _acc():
        o_ref[...] += contrib


def _experts(perm, w1b, w2b):
    return pl.pallas_call(
        _ffn_body,
        grid=(_E, _FT),
        in_specs=[
            pl.BlockSpec((_CHUNK, _DIM), lambda e, f: (e, 0)),
            pl.BlockSpec((1, _FTS, _DIM), lambda e, f: (e, f, 0)),
            pl.BlockSpec((1, _DIM, _FTS), lambda e, f: (e, 0, f)),
        ],
        out_specs=pl.BlockSpec((_CHUNK, _DIM), lambda e, f: (e, 0)),
        out_shape=jax.ShapeDtypeStruct((_S, _DIM), jnp.float32),
    )(perm, w1b, w2b)


def _unperm_body(y_hbm, pos_hbm, w_hbm, out_hbm,
                 pos_0, pos_1, pos_2, pos_3, w_v,
                 rows_a, rows_b, out_v, sem_a, sem_b):
    nc = 2
    wid = lax.axis_index("s") * nc + lax.axis_index("c")
    pos_refs = [pos_0, pos_1, pos_2, pos_3]
    for c in range(4):
        pltpu.sync_copy(pos_hbm.at[pl.ds(wid * _PER_W + c * 64, 64)], pos_refs[c])
    pltpu.sync_copy(w_hbm.at[pl.ds(wid * _PER_W, _PER_W)], w_v)
    iota16 = lax.broadcasted_iota(jnp.int32, (16,), 0)
    nchunk = _PER_W // 64
    g = pltpu.async_copy(y_hbm.at[pos_refs[0]], rows_a, sem_a)
    for c in range(nchunk):
        rows = rows_a if c % 2 == 0 else rows_b
        sem = sem_a if c % 2 == 0 else sem_b
        g.wait()
        if c + 1 < nchunk:
            nrows = rows_b if c % 2 == 0 else rows_a
            nsem = sem_b if c % 2 == 0 else sem_a
            g = pltpu.async_copy(y_hbm.at[pos_refs[c + 1]], nrows, nsem)
        def pair(i, _):
            w0 = plsc.load_gather(w_v, [jnp.full((16,), c * 64, jnp.int32) + 2 * i])
            w1 = plsc.load_gather(w_v, [jnp.full((16,), c * 64 + 1, jnp.int32) + 2 * i])
            r0 = jnp.full((16,), 0, jnp.int32) + 2 * i
            r1 = r0 + 1
            ri = jnp.full((16,), 0, jnp.int32) + i
            for q in range(_DIM // 16):
                colq = q * 16 + iota16
                a = plsc.load_gather(rows, [r0, colq])
                b = plsc.load_gather(rows, [r1, colq])
                plsc.store_scatter(out_v, [ri, colq], w0 * a + w1 * b)
            return 0

        lax.fori_loop(0, 32, pair, 0)
        pltpu.sync_copy(out_v, out_hbm.at[pl.ds(wid * _TOK_W + c * 32, 32)])


def _unpermute(y, pos, wts):
    mesh = plsc.VectorSubcoreMesh(core_axis_name="c", subcore_axis_name="s")
    fn = pl.kernel(
        _unperm_body,
        compiler_params=pltpu.CompilerParams(needs_layout_passes=False),
        out_type=jax.ShapeDtypeStruct((_N, _DIM), jnp.float32),
        mesh=mesh,
        scratch_types=[
            pltpu.VMEM((64,), jnp.int32),
            pltpu.VMEM((64,), jnp.int32),
            pltpu.VMEM((64,), jnp.int32),
            pltpu.VMEM((64,), jnp.int32),
            pltpu.VMEM((_PER_W,), jnp.float32),
            pltpu.VMEM((64, _DIM), jnp.float32),
            pltpu.VMEM((64, _DIM), jnp.float32),
            pltpu.VMEM((32, _DIM), jnp.float32),
            pltpu.SemaphoreType.DMA,
            pltpu.SemaphoreType.DMA,
        ],
    )
    return fn(y, pos, wts)


def kernel(x, router_W, w1, w2):
    x_flat = x.reshape(-1, _DIM)
    idx2, wts2, base = _router(x_flat, router_W)
    ids_flat = idx2.reshape(-1)
    pos = _rank(ids_flat, base)
    x_bf = x_flat.astype(jnp.bfloat16)
    x_i32 = lax.bitcast_convert_type(
        x_bf.reshape(_N, _DIM // 2, 2), jnp.int32)
    perm_i32 = _gather_sorted(x_i32, pos)
    perm = lax.bitcast_convert_type(perm_i32, jnp.bfloat16).reshape(_S, _DIM)
    y = _experts(perm, w1.astype(jnp.bfloat16), w2.astype(jnp.bfloat16))
    out = _unpermute(y, pos, wts2.reshape(-1))
    return out.reshape(_B, _T, _DIM)


# speedup vs baseline: 1.6603x; 1.6603x over previous
import functools

import jax
import jax.numpy as jnp
from jax import lax
from jax.experimental import pallas as pl
from jax.experimental.pallas import tpu as pltpu
from jax.experimental.pallas import tpu_sc as plsc

_B, _T, _DIM = 2, 2048, 768
_FF = 3072
_E = 8
_TOPK = 2
_N = _B * _T
_S = _N * _TOPK
_CHUNK = _S // _E
_NW = 32
_PER_W = _S // _NW
_TOK_W = _N // _NW
_FT = 2
_FTS = _FF // _FT


def _router_body(x_ref, rw_ref, idx_ref, wts_ref, base_ref):
    x = x_ref[...]
    rw = rw_ref[...]
    logits = lax.dot_general(x, rw, (((1,), (1,)), ((), ())),
                             preferred_element_type=jnp.float32)
    col = lax.broadcasted_iota(jnp.int32, (_N, _E), 1)
    m1 = jnp.max(logits, axis=1, keepdims=True)
    i1 = jnp.min(jnp.where(logits == m1, col, _E), axis=1, keepdims=True)
    masked = jnp.where(col == i1, -jnp.inf, logits)
    m2 = jnp.max(masked, axis=1, keepdims=True)
    i2 = jnp.min(jnp.where(masked == m2, col, _E), axis=1, keepdims=True)
    e21 = jnp.exp(m2 - m1)
    denom = 1.0 + e21
    idx_ref[...] = jnp.concatenate([i1, i2], axis=1)
    wts_ref[...] = jnp.concatenate([1.0 / denom, e21 / denom], axis=1)

    m_hits = ((i1 == col).astype(jnp.float32)
              + (i2 == col).astype(jnp.float32))
    r_row = lax.broadcasted_iota(jnp.int32, (_NW, _N), 0)
    r_tok = lax.broadcasted_iota(jnp.int32, (_NW, _N), 1)
    rsel = (r_tok // _TOK_W == r_row).astype(jnp.float32)
    counts = jnp.dot(rsel, m_hits, preferred_element_type=jnp.float32)
    tri_a = lax.broadcasted_iota(jnp.int32, (_NW, _NW), 0)
    tri_b = lax.broadcasted_iota(jnp.int32, (_NW, _NW), 1)
    tril_strict = (tri_b < tri_a).astype(jnp.float32)
    prefix = jnp.dot(tril_strict, counts, preferred_element_type=jnp.float32,
                     precision=lax.Precision.HIGHEST)
    totals = jnp.sum(counts, axis=0, keepdims=True)
    u_a = lax.broadcasted_iota(jnp.int32, (_E, _E), 0)
    u_b = lax.broadcasted_iota(jnp.int32, (_E, _E), 1)
    u_strict = (u_a < u_b).astype(jnp.float32)
    offs = jnp.dot(totals, u_strict, preferred_element_type=jnp.float32,
                   precision=lax.Precision.HIGHEST)
    base = (prefix + offs).astype(jnp.int32)
    base_ref[...] = jnp.concatenate(
        [base, jnp.zeros((_NW, _E), jnp.int32)], axis=1)


def _router(x_flat, router_W):
    return pl.pallas_call(
        _router_body,
        out_shape=(
            jax.ShapeDtypeStruct((_N, _TOPK), jnp.int32),
            jax.ShapeDtypeStruct((_N, _TOPK), jnp.float32),
            jax.ShapeDtypeStruct((_NW, 16), jnp.int32),
        ),
    )(x_flat, router_W)


def _gather_sorted_body(x_hbm, ids_hbm, base_hbm, perm_hbm, pos_hbm,
                        ids_v, base_v, pos_seg, pos_sh,
                        pos_all, src_all, src_0, src_1, src_2, src_3,
                        rows_a, rows_b, gsem, ssem):
    nc = 2
    cid = lax.axis_index("c")
    sid = lax.axis_index("s")
    wid = sid * nc + cid
    src_refs = [src_0, src_1, src_2, src_3]
    pltpu.sync_copy(ids_hbm.at[pl.ds(sid * 2 * _PER_W, 2 * _PER_W)], ids_v)
    for half in range(2):
        pltpu.sync_copy(base_hbm.at[sid * 2 + half], base_v)
        cnt = [jnp.int32(0)] * _E
        for v in range(_PER_W // 16):
            iv = ids_v[pl.ds(half * _PER_W + v * 16, 16)]
            rank = jnp.zeros((16,), jnp.int32)
            for e in range(_E):
                m = iv == e
                ones = jnp.where(m, 1, 0).astype(jnp.int32)
                csum = plsc.cumsum(ones)
                rank = jnp.where(m, csum - 1 + cnt[e], rank)
                cnt[e] = cnt[e] + jnp.sum(ones)
            pos_seg[pl.ds(half * _PER_W + v * 16, 16)] = (
                plsc.load_gather(base_v, [iv]) + rank)
    pltpu.sync_copy(pos_seg, pos_sh.at[pl.ds(sid * 2 * _PER_W, 2 * _PER_W)])

    @pl.when(cid == 0)
    def _write_pos():
        pltpu.sync_copy(pos_seg, pos_hbm.at[pl.ds(sid * 2 * _PER_W, 2 * _PER_W)])

    plsc.subcore_barrier()
    pltpu.sync_copy(pos_sh, pos_all)
    iota16 = lax.broadcasted_iota(jnp.int32, (16,), 0)
    lo = wid * _PER_W

    def scan(q, _):
        pos16 = plsc.load_gather(pos_all, [q * 16 + iota16])
        tok16 = lax.shift_right_arithmetic(q * 16 + iota16, 1)
        plsc.store_scatter(src_all, [pos16], tok16)
        return 0

    lax.fori_loop(0, _S // 16, scan, 0)
    for cc in range(4):
        for k in range(4):
            idxs = lo + cc * 64 + k * 16 + iota16
            vals = plsc.load_gather(src_all, [idxs])
            src_refs[cc][pl.ds(k * 16, 16)] = jnp.clip(vals, 0, _N - 1)
    nchunk = _PER_W // 64
    g = pltpu.async_copy(x_hbm.at[src_refs[0]], rows_a, gsem)
    for c in range(nchunk):
        buf = rows_a if c % 2 == 0 else rows_b
        g.wait()
        if c + 1 < nchunk:
            nbuf = rows_b if c % 2 == 0 else rows_a
            g = pltpu.async_copy(x_hbm.at[src_refs[c + 1]], nbuf, gsem)
        s = pltpu.async_copy(buf, perm_hbm.at[pl.ds(lo + c * 64, 64)], ssem)
        s.wait()


def _gather_sorted(x_flat, ids_flat, base):
    mesh = plsc.VectorSubcoreMesh(core_axis_name="c", subcore_axis_name="s")
    fn = pl.kernel(
        _gather_sorted_body,
        compiler_params=pltpu.CompilerParams(needs_layout_passes=False),
        out_type=(
            jax.ShapeDtypeStruct((_S, _DIM), jnp.float32),
            jax.ShapeDtypeStruct((_S,), jnp.int32),
        ),
        mesh=mesh,
        scratch_types=[
            pltpu.VMEM((2 * _PER_W,), jnp.int32),
            pltpu.VMEM((16,), jnp.int32),
            pltpu.VMEM((2 * _PER_W,), jnp.int32),
            pltpu.VMEM_SHARED((_S,), jnp.int32),
            pltpu.VMEM((_S,), jnp.int32),
            pltpu.VMEM((_S,), jnp.int32),
            pltpu.VMEM((64,), jnp.int32),
            pltpu.VMEM((64,), jnp.int32),
            pltpu.VMEM((64,), jnp.int32),
            pltpu.VMEM((64,), jnp.int32),
            pltpu.VMEM((64, _DIM), jnp.float32),
            pltpu.VMEM((64, _DIM), jnp.float32),
            pltpu.SemaphoreType.DMA,
            pltpu.SemaphoreType.DMA,
        ],
    )
    return fn(x_flat, ids_flat, base)


def _ffn_body(xc_ref, w1_ref, w2_ref, o_ref):
    f = pl.program_id(1)
    xb = xc_ref[...].astype(jnp.bfloat16)
    w1b = w1_ref[0]
    h = lax.dot_general(xb, w1b, (((1,), (1,)), ((), ())),
                        preferred_element_type=jnp.float32)
    h = 0.5 * h * (1.0 + jnp.tanh(0.7978845608028654 * (h + 0.044715 * h * h * h)))
    w2b = w2_ref[0]
    contrib = lax.dot_general(h.astype(jnp.bfloat16), w2b,
                              (((1,), (1,)), ((), ())),
                              preferred_element_type=jnp.float32)

    @pl.when(f == 0)
    def _init():
        o_ref[...] = contrib

    @pl.when(f != 0)
    def _acc():
        o_ref[...] += contrib


def _experts(perm, w1b, w2b):
    return pl.pallas_call(
        _ffn_body,
        grid=(_E, _FT),
        in_specs=[
            pl.BlockSpec((_CHUNK, _DIM), lambda e, f: (e, 0)),
            pl.BlockSpec((1, _FTS, _DIM), lambda e, f: (e, f, 0)),
            pl.BlockSpec((1, _DIM, _FTS), lambda e, f: (e, 0, f)),
        ],
        out_specs=pl.BlockSpec((_CHUNK, _DIM), lambda e, f: (e, 0)),
        out_shape=jax.ShapeDtypeStruct((_S, _DIM), jnp.float32),
    )(perm, w1b, w2b)


def _unperm_body(y_hbm, pos_hbm, w_hbm, out_hbm,
                 pos_0, pos_1, pos_2, pos_3, w_v,
                 rows_a, rows_b, out_v, sem_a, sem_b):
    nc = 2
    wid = lax.axis_index("s") * nc + lax.axis_index("c")
    pos_refs = [pos_0, pos_1, pos_2, pos_3]
    for c in range(4):
        pltpu.sync_copy(pos_hbm.at[pl.ds(wid * _PER_W + c * 64, 64)], pos_refs[c])
    pltpu.sync_copy(w_hbm.at[pl.ds(wid * _PER_W, _PER_W)], w_v)
    iota16 = lax.broadcasted_iota(jnp.int32, (16,), 0)
    nchunk = _PER_W // 64
    g = pltpu.async_copy(y_hbm.at[pos_refs[0]], rows_a, sem_a)
    for c in range(nchunk):
        rows = rows_a if c % 2 == 0 else rows_b
        sem = sem_a if c % 2 == 0 else sem_b
        g.wait()
        if c + 1 < nchunk:
            nrows = rows_b if c % 2 == 0 else rows_a
            nsem = sem_b if c % 2 == 0 else sem_a
            g = pltpu.async_copy(y_hbm.at[pos_refs[c + 1]], nrows, nsem)
        def pair(i, _):
            w0 = plsc.load_gather(w_v, [jnp.full((16,), c * 64, jnp.int32) + 2 * i])
            w1 = plsc.load_gather(w_v, [jnp.full((16,), c * 64 + 1, jnp.int32) + 2 * i])
            r0 = jnp.full((16,), 0, jnp.int32) + 2 * i
            r1 = r0 + 1
            ri = jnp.full((16,), 0, jnp.int32) + i
            for q in range(_DIM // 16):
                colq = q * 16 + iota16
                a = plsc.load_gather(rows, [r0, colq])
                b = plsc.load_gather(rows, [r1, colq])
                plsc.store_scatter(out_v, [ri, colq], w0 * a + w1 * b)
            return 0

        lax.fori_loop(0, 32, pair, 0)
        pltpu.sync_copy(out_v, out_hbm.at[pl.ds(wid * _TOK_W + c * 32, 32)])


def _unpermute(y, pos, wts):
    mesh = plsc.VectorSubcoreMesh(core_axis_name="c", subcore_axis_name="s")
    fn = pl.kernel(
        _unperm_body,
        compiler_params=pltpu.CompilerParams(needs_layout_passes=False),
        out_type=jax.ShapeDtypeStruct((_N, _DIM), jnp.float32),
        mesh=mesh,
        scratch_types=[
            pltpu.VMEM((64,), jnp.int32),
            pltpu.VMEM((64,), jnp.int32),
            pltpu.VMEM((64,), jnp.int32),
            pltpu.VMEM((64,), jnp.int32),
            pltpu.VMEM((_PER_W,), jnp.float32),
            pltpu.VMEM((64, _DIM), jnp.float32),
            pltpu.VMEM((64, _DIM), jnp.float32),
            pltpu.VMEM((32, _DIM), jnp.float32),
            pltpu.SemaphoreType.DMA,
            pltpu.SemaphoreType.DMA,
        ],
    )
    return fn(y, pos, wts)


def kernel(x, router_W, w1, w2):
    x_flat = x.reshape(-1, _DIM)
    idx2, wts2, base = _router(x_flat, router_W)
    ids_flat = idx2.reshape(-1)
    perm, pos = _gather_sorted(x_flat, ids_flat, base)
    y = _experts(perm, w1.astype(jnp.bfloat16), w2.astype(jnp.bfloat16))
    out = _unpermute(y, pos, wts2.reshape(-1))
    return out.reshape(_B, _T, _DIM)


# speedup vs baseline: 1.6976x; 1.0225x over previous
import functools

import jax
import jax.numpy as jnp
from jax import lax
from jax.experimental import pallas as pl
from jax.experimental.pallas import tpu as pltpu
from jax.experimental.pallas import tpu_sc as plsc

_B, _T, _DIM = 2, 2048, 768
_FF = 3072
_E = 8
_TOPK = 2
_N = _B * _T
_S = _N * _TOPK
_CHUNK = _S // _E
_NW = 32
_PER_W = _S // _NW
_TOK_W = _N // _NW
_FT = 1
_FTS = _FF // _FT


def _router_body(x_ref, rw_ref, idx_ref, wts_ref, base_ref):
    x = x_ref[...]
    rw = rw_ref[...]
    logits = lax.dot_general(x, rw, (((1,), (1,)), ((), ())),
                             preferred_element_type=jnp.float32)
    col = lax.broadcasted_iota(jnp.int32, (_N, _E), 1)
    m1 = jnp.max(logits, axis=1, keepdims=True)
    i1 = jnp.min(jnp.where(logits == m1, col, _E), axis=1, keepdims=True)
    masked = jnp.where(col == i1, -jnp.inf, logits)
    m2 = jnp.max(masked, axis=1, keepdims=True)
    i2 = jnp.min(jnp.where(masked == m2, col, _E), axis=1, keepdims=True)
    e21 = jnp.exp(m2 - m1)
    denom = 1.0 + e21
    idx_ref[...] = jnp.concatenate([i1, i2], axis=1)
    wts_ref[...] = jnp.concatenate([1.0 / denom, e21 / denom], axis=1)

    m_hits = ((i1 == col).astype(jnp.float32)
              + (i2 == col).astype(jnp.float32))
    r_row = lax.broadcasted_iota(jnp.int32, (_NW, _N), 0)
    r_tok = lax.broadcasted_iota(jnp.int32, (_NW, _N), 1)
    rsel = (r_tok // _TOK_W == r_row).astype(jnp.float32)
    counts = jnp.dot(rsel, m_hits, preferred_element_type=jnp.float32)
    tri_a = lax.broadcasted_iota(jnp.int32, (_NW, _NW), 0)
    tri_b = lax.broadcasted_iota(jnp.int32, (_NW, _NW), 1)
    tril_strict = (tri_b < tri_a).astype(jnp.float32)
    prefix = jnp.dot(tril_strict, counts, preferred_element_type=jnp.float32,
                     precision=lax.Precision.HIGHEST)
    totals = jnp.sum(counts, axis=0, keepdims=True)
    u_a = lax.broadcasted_iota(jnp.int32, (_E, _E), 0)
    u_b = lax.broadcasted_iota(jnp.int32, (_E, _E), 1)
    u_strict = (u_a < u_b).astype(jnp.float32)
    offs = jnp.dot(totals, u_strict, preferred_element_type=jnp.float32,
                   precision=lax.Precision.HIGHEST)
    base = (prefix + offs).astype(jnp.int32)
    base_ref[...] = jnp.concatenate(
        [base, jnp.zeros((_NW, _E), jnp.int32)], axis=1)


def _router(x_flat, router_W):
    return pl.pallas_call(
        _router_body,
        out_shape=(
            jax.ShapeDtypeStruct((_N, _TOPK), jnp.int32),
            jax.ShapeDtypeStruct((_N, _TOPK), jnp.float32),
            jax.ShapeDtypeStruct((_NW, 16), jnp.int32),
        ),
    )(x_flat, router_W)


def _gather_sorted_body(x_hbm, ids_hbm, base_hbm, perm_hbm, pos_hbm,
                        ids_v, base_v, pos_seg, pos_sh,
                        pos_all, src_all, src_0, src_1, src_2, src_3,
                        rows_a, rows_b, gsem, ssem):
    nc = 2
    cid = lax.axis_index("c")
    sid = lax.axis_index("s")
    wid = sid * nc + cid
    src_refs = [src_0, src_1, src_2, src_3]
    pltpu.sync_copy(ids_hbm.at[pl.ds(sid * 2 * _PER_W, 2 * _PER_W)], ids_v)
    for half in range(2):
        pltpu.sync_copy(base_hbm.at[sid * 2 + half], base_v)
        cnt = [jnp.int32(0)] * _E
        for v in range(_PER_W // 16):
            iv = ids_v[pl.ds(half * _PER_W + v * 16, 16)]
            rank = jnp.zeros((16,), jnp.int32)
            for e in range(_E):
                m = iv == e
                ones = jnp.where(m, 1, 0).astype(jnp.int32)
                csum = plsc.cumsum(ones)
                rank = jnp.where(m, csum - 1 + cnt[e], rank)
                cnt[e] = cnt[e] + jnp.sum(ones)
            pos_seg[pl.ds(half * _PER_W + v * 16, 16)] = (
                plsc.load_gather(base_v, [iv]) + rank)
    pltpu.sync_copy(pos_seg, pos_sh.at[pl.ds(sid * 2 * _PER_W, 2 * _PER_W)])

    @pl.when(cid == 0)
    def _write_pos():
        pltpu.sync_copy(pos_seg, pos_hbm.at[pl.ds(sid * 2 * _PER_W, 2 * _PER_W)])

    plsc.subcore_barrier()
    pltpu.sync_copy(pos_sh, pos_all)
    iota16 = lax.broadcasted_iota(jnp.int32, (16,), 0)
    lo = wid * _PER_W

    def scan(q, _):
        pos16 = plsc.load_gather(pos_all, [q * 16 + iota16])
        tok16 = lax.shift_right_arithmetic(q * 16 + iota16, 1)
        plsc.store_scatter(src_all, [pos16], tok16)
        return 0

    lax.fori_loop(0, _S // 16, scan, 0)
    for cc in range(4):
        for k in range(4):
            idxs = lo + cc * 64 + k * 16 + iota16
            vals = plsc.load_gather(src_all, [idxs])
            src_refs[cc][pl.ds(k * 16, 16)] = jnp.clip(vals, 0, _N - 1)
    nchunk = _PER_W // 64
    g = pltpu.async_copy(x_hbm.at[src_refs[0]], rows_a, gsem)
    for c in range(nchunk):
        buf = rows_a if c % 2 == 0 else rows_b
        g.wait()
        if c + 1 < nchunk:
            nbuf = rows_b if c % 2 == 0 else rows_a
            g = pltpu.async_copy(x_hbm.at[src_refs[c + 1]], nbuf, gsem)
        s = pltpu.async_copy(buf, perm_hbm.at[pl.ds(lo + c * 64, 64)], ssem)
        s.wait()


def _gather_sorted(x_flat, ids_flat, base):
    mesh = plsc.VectorSubcoreMesh(core_axis_name="c", subcore_axis_name="s")
    fn = pl.kernel(
        _gather_sorted_body,
        compiler_params=pltpu.CompilerParams(needs_layout_passes=False),
        out_type=(
            jax.ShapeDtypeStruct((_S, _DIM), jnp.float32),
            jax.ShapeDtypeStruct((_S,), jnp.int32),
        ),
        mesh=mesh,
        scratch_types=[
            pltpu.VMEM((2 * _PER_W,), jnp.int32),
            pltpu.VMEM((16,), jnp.int32),
            pltpu.VMEM((2 * _PER_W,), jnp.int32),
            pltpu.VMEM_SHARED((_S,), jnp.int32),
            pltpu.VMEM((_S,), jnp.int32),
            pltpu.VMEM((_S,), jnp.int32),
            pltpu.VMEM((64,), jnp.int32),
            pltpu.VMEM((64,), jnp.int32),
            pltpu.VMEM((64,), jnp.int32),
            pltpu.VMEM((64,), jnp.int32),
            pltpu.VMEM((64, _DIM), jnp.float32),
            pltpu.VMEM((64, _DIM), jnp.float32),
            pltpu.SemaphoreType.DMA,
            pltpu.SemaphoreType.DMA,
        ],
    )
    return fn(x_flat, ids_flat, base)


def _ffn_body(xc_ref, w1_ref, w2_ref, o_ref):
    f = pl.program_id(1)
    xb = xc_ref[...].astype(jnp.bfloat16)
    w1b = w1_ref[0]
    h = lax.dot_general(xb, w1b, (((1,), (1,)), ((), ())),
                        preferred_element_type=jnp.float32)
    h = 0.5 * h * (1.0 + jnp.tanh(0.7978845608028654 * (h + 0.044715 * h * h * h)))
    w2b = w2_ref[0]
    contrib = lax.dot_general(h.astype(jnp.bfloat16), w2b,
                              (((1,), (1,)), ((), ())),
                              preferred_element_type=jnp.float32)

    @pl.when(f == 0)
    def _init():
        o_ref[...] = contrib

    @pl.when(f != 0)
    def _acc():
        o_ref[...] += contrib


def _experts(perm, w1b, w2b):
    return pl.pallas_call(
        _ffn_body,
        grid=(_E, _FT),
        in_specs=[
            pl.BlockSpec((_CHUNK, _DIM), lambda e, f: (e, 0)),
            pl.BlockSpec((1, _FTS, _DIM), lambda e, f: (e, f, 0)),
            pl.BlockSpec((1, _DIM, _FTS), lambda e, f: (e, 0, f)),
        ],
        out_specs=pl.BlockSpec((_CHUNK, _DIM), lambda e, f: (e, 0)),
        out_shape=jax.ShapeDtypeStruct((_S, _DIM), jnp.float32),
    )(perm, w1b, w2b)


def _unperm_body(y_hbm, pos_hbm, w_hbm, out_hbm,
                 pos_0, pos_1, pos_2, pos_3, w_v,
                 rows_a, rows_b, out_v, sem_a, sem_b):
    nc = 2
    wid = lax.axis_index("s") * nc + lax.axis_index("c")
    pos_refs = [pos_0, pos_1, pos_2, pos_3]
    for c in range(4):
        pltpu.sync_copy(pos_hbm.at[pl.ds(wid * _PER_W + c * 64, 64)], pos_refs[c])
    pltpu.sync_copy(w_hbm.at[pl.ds(wid * _PER_W, _PER_W)], w_v)
    iota16 = lax.broadcasted_iota(jnp.int32, (16,), 0)
    nchunk = _PER_W // 64
    g = pltpu.async_copy(y_hbm.at[pos_refs[0]], rows_a, sem_a)
    for c in range(nchunk):
        rows = rows_a if c % 2 == 0 else rows_b
        sem = sem_a if c % 2 == 0 else sem_b
        g.wait()
        if c + 1 < nchunk:
            nrows = rows_b if c % 2 == 0 else rows_a
            nsem = sem_b if c % 2 == 0 else sem_a
            g = pltpu.async_copy(y_hbm.at[pos_refs[c + 1]], nrows, nsem)
        def pair(i, _):
            w0 = plsc.load_gather(w_v, [jnp.full((16,), c * 64, jnp.int32) + 2 * i])
            w1 = plsc.load_gather(w_v, [jnp.full((16,), c * 64 + 1, jnp.int32) + 2 * i])
            r0 = jnp.full((16,), 0, jnp.int32) + 2 * i
            r1 = r0 + 1
            ri = jnp.full((16,), 0, jnp.int32) + i
            for q in range(_DIM // 16):
                colq = q * 16 + iota16
                a = plsc.load_gather(rows, [r0, colq])
                b = plsc.load_gather(rows, [r1, colq])
                plsc.store_scatter(out_v, [ri, colq], w0 * a + w1 * b)
            return 0

        lax.fori_loop(0, 32, pair, 0)
        pltpu.sync_copy(out_v, out_hbm.at[pl.ds(wid * _TOK_W + c * 32, 32)])


def _unpermute(y, pos, wts):
    mesh = plsc.VectorSubcoreMesh(core_axis_name="c", subcore_axis_name="s")
    fn = pl.kernel(
        _unperm_body,
        compiler_params=pltpu.CompilerParams(needs_layout_passes=False),
        out_type=jax.ShapeDtypeStruct((_N, _DIM), jnp.float32),
        mesh=mesh,
        scratch_types=[
            pltpu.VMEM((64,), jnp.int32),
            pltpu.VMEM((64,), jnp.int32),
            pltpu.VMEM((64,), jnp.int32),
            pltpu.VMEM((64,), jnp.int32),
            pltpu.VMEM((_PER_W,), jnp.float32),
            pltpu.VMEM((64, _DIM), jnp.float32),
            pltpu.VMEM((64, _DIM), jnp.float32),
            pltpu.VMEM((32, _DIM), jnp.float32),
            pltpu.SemaphoreType.DMA,
            pltpu.SemaphoreType.DMA,
        ],
    )
    return fn(y, pos, wts)


def kernel(x, router_W, w1, w2):
    x_flat = x.reshape(-1, _DIM)
    idx2, wts2, base = _router(x_flat, router_W)
    ids_flat = idx2.reshape(-1)
    perm, pos = _gather_sorted(x_flat, ids_flat, base)
    y = _experts(perm, w1.astype(jnp.bfloat16), w2.astype(jnp.bfloat16))
    out = _unpermute(y, pos, wts2.reshape(-1))
    return out.reshape(_B, _T, _DIM)
